# Initial kernel scaffold; baseline (speedup 1.0000x reference)
import jax
import jax.numpy as jnp
from jax import lax
from jax.experimental import pallas as pl
from jax.experimental.pallas import tpu as pltpu
from jax.experimental.pallas import tpu_sc as plsc

NC = 2
NS = 16
L = 16

N = 10000
E = 160000
F = 256
FH = F // NC
N_PAD = 10240
ROWS_PER_TILE = N_PAD // NS

EDGES_PER_TILE = E // NS
CHUNK = 400
DMA_B = 80
NB = CHUNK // DMA_B
NCHUNKS = EDGES_PER_TILE // CHUNK


def _mlp_body(x_ref, w1_ref, b1_ref, w2_ref, b2_ref, w3_ref, b3_ref,
              h_ref, hsplit_ref):
    h = jnp.dot(x_ref[...], w1_ref[...], preferred_element_type=jnp.float32)
    h = jax.nn.relu(h + b1_ref[...])
    h = jnp.dot(h, w2_ref[...], preferred_element_type=jnp.float32)
    h = jax.nn.relu(h + b2_ref[...])
    h = jnp.dot(h, w3_ref[...], preferred_element_type=jnp.float32)
    h = h + b3_ref[...]
    h_ref[...] = h
    hsplit_ref[...] = jnp.stack([h[:, :FH], h[:, FH:]], axis=0)


def _mlp(x, W1, b1, W2, b2, W3, b3):
    R = 1000
    grid = N // R
    return pl.pallas_call(
        _mlp_body,
        grid=(grid,),
        in_specs=[
            pl.BlockSpec((R, F), lambda i: (i, 0)),
            pl.BlockSpec((F, F), lambda i: (0, 0)),
            pl.BlockSpec((1, F), lambda i: (0, 0)),
            pl.BlockSpec((F, F), lambda i: (0, 0)),
            pl.BlockSpec((1, F), lambda i: (0, 0)),
            pl.BlockSpec((F, F), lambda i: (0, 0)),
            pl.BlockSpec((1, F), lambda i: (0, 0)),
        ],
        out_specs=[
            pl.BlockSpec((R, F), lambda i: (i, 0)),
            pl.BlockSpec((NC, R, FH), lambda i: (0, i, 0)),
        ],
        out_shape=[
            jax.ShapeDtypeStruct((N, F), jnp.float32),
            jax.ShapeDtypeStruct((NC, N, FH), jnp.float32),
        ],
    )(x, W1, b1.reshape(1, F), W2, b2.reshape(1, F), W3, b3.reshape(1, F))


def _sc_agg_body(hsplit_hbm, src_hbm, dst_hbm,
                 summed_hbm, cnt_hbm,
                 acc, cnt_sh,
                 srcb, dstb, rows, cnt_local, cbuf, tbuf, sem):
    c = lax.axis_index("c")
    s = lax.axis_index("s")
    zeros16 = jnp.zeros((L,), jnp.float32)
    ones16 = jnp.ones((L,), jnp.float32)

    @pl.loop(0, 64)
    def _(i):
        for col in range(FH // L):
            rows[i, pl.ds(col * L, L)] = zeros16

    for j in range(ROWS_PER_TILE // 64):
        pltpu.sync_copy(rows.at[pl.ds(0, 64), :],
                        acc.at[pl.ds(s * ROWS_PER_TILE + j * 64, 64), :])

    @pl.loop(0, N_PAD // L)
    def _(i):
        cnt_local[pl.ds(i * L, L)] = zeros16

    plsc.subcore_barrier()

    base = s * EDGES_PER_TILE
    src_off = c * E

    @pl.loop(0, NCHUNKS)
    def _(g):
        off = base + g * CHUNK
        for j in range(NB):
            pltpu.sync_copy(src_hbm.at[pl.ds(src_off + off + j * DMA_B, DMA_B)],
                            srcb.at[j])
            pltpu.sync_copy(dst_hbm.at[pl.ds(off + j * DMA_B, DMA_B)],
                            dstb.at[j])
        descs = []
        for j in range(NB):
            descs.append(pltpu.async_copy(
                hsplit_hbm.at[srcb.at[j]],
                rows.at[pl.ds(j * DMA_B, DMA_B), :],
                sem))
        for d in descs:
            d.wait()
        for j in range(NB):
            pltpu.sync_copy(rows.at[pl.ds(j * DMA_B, DMA_B), :],
                            acc.at[dstb.at[j]], add=True)
        for j in range(NB):
            for k in range(DMA_B // L):
                dv = dstb[j, pl.ds(k * L, L)]
                plsc.addupdate_scatter(cnt_local, [dv], ones16)

    plsc.subcore_barrier()

    pltpu.sync_copy(cnt_local, cnt_sh.at[s])
    plsc.subcore_barrier()

    col = s * ROWS_PER_TILE
    pltpu.sync_copy(cnt_sh.at[0, pl.ds(col, ROWS_PER_TILE)], cbuf)

    @pl.loop(1, NS)
    def _(t):
        pltpu.sync_copy(cnt_sh.at[t, pl.ds(col, ROWS_PER_TILE)], tbuf)

        @pl.loop(0, ROWS_PER_TILE // L)
        def _(k):
            cbuf[pl.ds(k * L, L)] = cbuf[pl.ds(k * L, L)] + tbuf[pl.ds(k * L, L)]

    @pl.when(c == 0)
    def _():
        pltpu.sync_copy(cbuf, cnt_hbm.at[pl.ds(col, ROWS_PER_TILE)])

    pltpu.sync_copy(
        acc.at[pl.ds(s * ROWS_PER_TILE, ROWS_PER_TILE), :],
        summed_hbm.at[pl.ds(c * N_PAD + s * ROWS_PER_TILE, ROWS_PER_TILE), :])


def _sc_agg(hsplit, src2, dst):
    mesh = plsc.VectorSubcoreMesh(core_axis_name="c", subcore_axis_name="s")
    return pl.kernel(
        _sc_agg_body,
        out_type=(
            jax.ShapeDtypeStruct((NC * N_PAD, FH), jnp.float32),
            jax.ShapeDtypeStruct((N_PAD,), jnp.float32),
        ),
        mesh=mesh,
        scratch_types=[
            pltpu.VMEM_SHARED((N_PAD, FH), jnp.float32),
            pltpu.VMEM_SHARED((NS, N_PAD), jnp.float32),
            pltpu.VMEM((NB, DMA_B), jnp.int32),
            pltpu.VMEM((NB, DMA_B), jnp.int32),
            pltpu.VMEM((CHUNK, FH), jnp.float32),
            pltpu.VMEM((N_PAD,), jnp.float32),
            pltpu.VMEM((ROWS_PER_TILE,), jnp.float32),
            pltpu.VMEM((ROWS_PER_TILE,), jnp.float32),
            pltpu.SemaphoreType.DMA,
        ],
    )(hsplit, src2, dst)


def _combine_body(h_ref, s_ref, cnt_ref, lin_ref, out_ref):
    cnt = jnp.maximum(cnt_ref[...], 1.0)
    mean = jnp.concatenate([s_ref[0] / cnt, s_ref[1] / cnt], axis=1)
    out_ref[...] = h_ref[...] + mean * lin_ref[...]


def _combine(h, summed, cnt, lin_l):
    R = 400
    grid = N // R
    return pl.pallas_call(
        _combine_body,
        grid=(grid,),
        in_specs=[
            pl.BlockSpec((R, F), lambda i: (i, 0)),
            pl.BlockSpec((NC, R, FH), lambda i: (0, i, 0)),
            pl.BlockSpec((R, 1), lambda i: (i, 0)),
            pl.BlockSpec((1, F), lambda i: (0, 0)),
        ],
        out_specs=pl.BlockSpec((R, F), lambda i: (i, 0)),
        out_shape=jax.ShapeDtypeStruct((N, F), jnp.float32),
    )(h, summed.reshape(NC, N_PAD, FH), cnt.reshape(N_PAD, 1),
      lin_l.reshape(1, F))


@jax.jit
def kernel(x, edge_index, W1, b1, W2, b2, W3, b3, lin_l):
    assert x.shape == (N, F) and edge_index.shape == (2, E)
    h, hsplit = _mlp(x, W1, b1, W2, b2, W3, b3)
    src = edge_index[0]
    dst = edge_index[1]
    src2 = jnp.concatenate([src, src + N])
    summed, cnt = _sc_agg(hsplit.reshape(NC * N, FH), src2, dst)
    return _combine(h, summed, cnt, lin_l)


# speedup vs baseline: 4.1756x; 4.1756x over previous
import jax
import jax.numpy as jnp
from jax import lax
from jax.experimental import pallas as pl
from jax.experimental.pallas import tpu as pltpu
from jax.experimental.pallas import tpu_sc as plsc

NC = 2
NS = 16
L = 16

N = 10000
E = 160000
F = 256
FH = F // NC
N_PAD = 10240
ROWS_PER_TILE = N_PAD // NS

CHUNK = 64
NCHUNKS = 160
GRP = 16
NGRP = NCHUNKS // GRP
NBUF = 4
EPT_PAD = CHUNK * NCHUNKS
E_PAD = NS * EPT_PAD


def _mlp_body(x_ref, w1_ref, b1_ref, w2_ref, b2_ref, w3_ref, b3_ref,
              h_ref, hsplit_ref):
    h = jnp.dot(x_ref[...], w1_ref[...], preferred_element_type=jnp.float32)
    h = jax.nn.relu(h + b1_ref[...])
    h = jnp.dot(h, w2_ref[...], preferred_element_type=jnp.float32)
    h = jax.nn.relu(h + b2_ref[...])
    h = jnp.dot(h, w3_ref[...], preferred_element_type=jnp.float32)
    h = h + b3_ref[...]
    h_ref[...] = h
    hsplit_ref[...] = jnp.stack([h[:, :FH], h[:, FH:]], axis=0)


def _mlp(x, W1, b1, W2, b2, W3, b3):
    R = 1000
    grid = N // R
    return pl.pallas_call(
        _mlp_body,
        grid=(grid,),
        in_specs=[
            pl.BlockSpec((R, F), lambda i: (i, 0)),
            pl.BlockSpec((F, F), lambda i: (0, 0)),
            pl.BlockSpec((1, F), lambda i: (0, 0)),
            pl.BlockSpec((F, F), lambda i: (0, 0)),
            pl.BlockSpec((1, F), lambda i: (0, 0)),
            pl.BlockSpec((F, F), lambda i: (0, 0)),
            pl.BlockSpec((1, F), lambda i: (0, 0)),
        ],
        out_specs=[
            pl.BlockSpec((R, F), lambda i: (i, 0)),
            pl.BlockSpec((NC, R, FH), lambda i: (0, i, 0)),
        ],
        out_shape=[
            jax.ShapeDtypeStruct((N, F), jnp.float32),
            jax.ShapeDtypeStruct((NC, N, FH), jnp.float32),
        ],
    )(x, W1, b1.reshape(1, F), W2, b2.reshape(1, F), W3, b3.reshape(1, F))


def _sc_agg_body(hsplit_hbm, src_hbm, dst_hbm,
                 summed_hbm, cntp_hbm,
                 acc,
                 srcb, dstb, rows, cnt_local,
                 sem0, sem1, sem2, sem3, ssem0, ssem1, ssem2, ssem3):
    c = lax.axis_index("c")
    s = lax.axis_index("s")
    zeros16 = jnp.zeros((L,), jnp.float32)
    ones16 = jnp.ones((L,), jnp.float32)

    @pl.loop(0, 64)
    def _(i):
        for col in range(FH // L):
            rows[0, i, pl.ds(col * L, L)] = zeros16

    @pl.loop(0, N_PAD // L)
    def _(i):
        cnt_local[pl.ds(i * L, L)] = zeros16

    for j in range(ROWS_PER_TILE // 64):
        pltpu.sync_copy(rows.at[0, pl.ds(0, 64), :],
                        acc.at[pl.ds(s * ROWS_PER_TILE + j * 64, 64), :])

    plsc.subcore_barrier()

    sems = (sem0, sem1, sem2, sem3)
    ssems = (ssem0, ssem1, ssem2, ssem3)

    @pl.loop(0, NGRP)
    def _(grp):
        pltpu.sync_copy(
            src_hbm.at[pl.ds((c * NS + s) * NCHUNKS + grp * GRP, GRP), :],
            srcb)
        pltpu.sync_copy(
            dst_hbm.at[pl.ds(s * NCHUNKS + grp * GRP, GRP), :], dstb)
        descs = [None] * NBUF
        sdescs = [None] * NBUF
        for t in range(NBUF - 1):
            descs[t] = pltpu.async_copy(hsplit_hbm.at[srcb.at[t]],
                                        rows.at[t], sems[t])
        for j in range(GRP):
            p = j % NBUF
            nxt = j + NBUF - 1
            if nxt < GRP:
                q = nxt % NBUF
                if sdescs[q] is not None:
                    sdescs[q].wait()
                    sdescs[q] = None
                descs[q] = pltpu.async_copy(
                    hsplit_hbm.at[srcb.at[nxt]], rows.at[q], sems[q])
            descs[p].wait()
            sdescs[p] = pltpu.async_copy(rows.at[p], acc.at[dstb.at[j]],
                                         ssems[p], add=True)
            for k in range(CHUNK // L):
                dv = dstb[j, pl.ds(k * L, L)]
                plsc.addupdate_scatter(cnt_local, [dv], ones16)
        for d in sdescs:
            if d is not None:
                d.wait()

    plsc.subcore_barrier()

    col = s * ROWS_PER_TILE

    @pl.when(c == 0)
    def _():
        pltpu.sync_copy(cnt_local, cntp_hbm.at[s])

    for j in range(ROWS_PER_TILE // CHUNK):
        pltpu.sync_copy(acc.at[pl.ds(col + j * CHUNK, CHUNK), :], rows.at[0])
        pltpu.sync_copy(
            rows.at[0],
            summed_hbm.at[pl.ds(c * N_PAD + col + j * CHUNK, CHUNK), :])


def _sc_agg(hsplit, src2, dst):
    mesh = plsc.VectorSubcoreMesh(core_axis_name="c", subcore_axis_name="s",
                                  num_cores=NC, num_subcores=NS)
    return pl.kernel(
        _sc_agg_body,
        out_type=(
            jax.ShapeDtypeStruct((NC * N_PAD, FH), jnp.float32),
            jax.ShapeDtypeStruct((NS, N_PAD), jnp.float32),
        ),
        mesh=mesh,
        scratch_types=[
            pltpu.VMEM_SHARED((N_PAD, FH), jnp.float32),
            pltpu.VMEM((GRP, CHUNK), jnp.int32),
            pltpu.VMEM((GRP, CHUNK), jnp.int32),
            pltpu.VMEM((NBUF, CHUNK, FH), jnp.float32),
            pltpu.VMEM((N_PAD,), jnp.float32),
            pltpu.SemaphoreType.DMA,
            pltpu.SemaphoreType.DMA,
            pltpu.SemaphoreType.DMA,
            pltpu.SemaphoreType.DMA,
            pltpu.SemaphoreType.DMA,
            pltpu.SemaphoreType.DMA,
            pltpu.SemaphoreType.DMA,
            pltpu.SemaphoreType.DMA,
        ],
        compiler_params=pltpu.CompilerParams(needs_layout_passes=False),
    )(hsplit, src2, dst)


def _combine_body(h_ref, s_ref, cnt_ref, lin_ref, out_ref):
    cnt = jnp.maximum(
        jnp.sum(cnt_ref[...], axis=1, keepdims=True), 1.0)
    mean = jnp.concatenate([s_ref[0] / cnt, s_ref[1] / cnt], axis=1)
    out_ref[...] = h_ref[...] + mean * lin_ref[...]


def _combine(h, summed, cnt, lin_l):
    R = 400
    grid = N // R
    return pl.pallas_call(
        _combine_body,
        grid=(grid,),
        in_specs=[
            pl.BlockSpec((R, F), lambda i: (i, 0)),
            pl.BlockSpec((NC, R, FH), lambda i: (0, i, 0)),
            pl.BlockSpec((R, NS), lambda i: (i, 0)),
            pl.BlockSpec((1, F), lambda i: (0, 0)),
        ],
        out_specs=pl.BlockSpec((R, F), lambda i: (i, 0)),
        out_shape=jax.ShapeDtypeStruct((N, F), jnp.float32),
    )(h, summed.reshape(NC, N_PAD, FH), cnt, lin_l.reshape(1, F))


@jax.jit
def kernel(x, edge_index, W1, b1, W2, b2, W3, b3, lin_l):
    assert x.shape == (N, F) and edge_index.shape == (2, E)
    h, hsplit = _mlp(x, W1, b1, W2, b2, W3, b3)
    pad = E_PAD - E
    src = jnp.concatenate([edge_index[0], jnp.zeros((pad,), jnp.int32)])
    dst = jnp.concatenate([edge_index[1],
                           jnp.full((pad,), N_PAD - 1, jnp.int32)])
    src2 = jnp.concatenate([src, src + N]).reshape(NC * NS * NCHUNKS, CHUNK)
    dst2 = dst.reshape(NS * NCHUNKS, CHUNK)
    summed, cnt_parts = _sc_agg(hsplit.reshape(NC * N, FH), src2, dst2)
    return _combine(h, summed, cnt_parts.T, lin_l)
